# baseline (speedup 1.0000x reference)
import functools

import jax
import jax.numpy as jnp
from jax import lax
from jax.experimental import pallas as pl
from jax.experimental.pallas import tpu as pltpu
from jax.experimental.pallas import tpu_sc as plsc

_NC = 2
_NS = 16
_NW = _NC * _NS
_CHUNK = 32
_LANES = 16


def _sc_body(x_hbm, w_hbm, o_hbm, bx, bw, sem):
    wid = lax.axis_index("s") * _NC + lax.axis_index("c")
    n_rows = x_hbm.shape[0]
    seq_len, d = w_hbm.shape
    vecs_per_row = d // _LANES
    rows_per_w = n_rows // _NW
    base = wid * rows_per_w
    for c in range(rows_per_w // _CHUNK):
        r0 = base + c * _CHUNK
        s0 = lax.rem(r0, seq_len)
        pltpu.sync_copy(x_hbm.at[pl.ds(r0, _CHUNK)], bx)
        pltpu.sync_copy(w_hbm.at[pl.ds(s0, _CHUNK)], bw)

        def row_add(r, carry):
            for j in range(vecs_per_row):
                sl = pl.ds(j * _LANES, _LANES)
                bx[r, sl] = bx[r, sl] + bw[r, sl]
            return carry

        lax.fori_loop(0, _CHUNK, row_add, 0)
        pltpu.sync_copy(bx, o_hbm.at[pl.ds(r0, _CHUNK)])


def kernel(x, encoding_weight):
    batch, seq_len, d_model = x.shape
    x2 = x.reshape(batch * seq_len, d_model)
    sc_add = functools.partial(
        pl.kernel,
        out_type=jax.ShapeDtypeStruct((batch * seq_len, d_model), x.dtype),
        mesh=plsc.VectorSubcoreMesh(core_axis_name="c", subcore_axis_name="s"),
        scratch_types=[
            pltpu.VMEM((_CHUNK, d_model), jnp.float32),
            pltpu.VMEM((_CHUNK, d_model), jnp.float32),
            pltpu.SemaphoreType.DMA,
        ],
    )(_sc_body)
    out = sc_add(x2, encoding_weight)
    return out.reshape(batch, seq_len, d_model)


# speedup vs baseline: 3.5593x; 3.5593x over previous
import jax
import jax.numpy as jnp
from jax import lax
from jax.experimental import pallas as pl


def _add_kernel(x_ref, w_ref, o_ref, *, rblk, seq_len):
    s0 = lax.rem(pl.program_id(0) * rblk, seq_len)
    o_ref[...] = x_ref[...] + w_ref[pl.ds(s0, rblk), :]


import functools


def kernel(x, encoding_weight):
    batch, seq_len, d_model = x.shape
    x2 = x.reshape(batch * seq_len, d_model)
    rblk = 1024
    n_rows = batch * seq_len
    grid = (n_rows // rblk,)
    out = pl.pallas_call(
        functools.partial(_add_kernel, rblk=rblk, seq_len=seq_len),
        grid=grid,
        in_specs=[
            pl.BlockSpec((rblk, d_model), lambda i: (i, 0)),
            pl.BlockSpec((seq_len, d_model), lambda i: (0, 0)),
        ],
        out_specs=pl.BlockSpec((rblk, d_model), lambda i: (i, 0)),
        out_shape=jax.ShapeDtypeStruct((n_rows, d_model), x.dtype),
    )(x2, encoding_weight)
    return out.reshape(batch, seq_len, d_model)


# speedup vs baseline: 3.6636x; 1.0293x over previous
import jax
import jax.numpy as jnp
from jax import lax
from jax.experimental import pallas as pl


def _add_kernel(x_ref, w_ref, o_ref, *, rblk, seq_len):
    s0 = lax.rem(pl.program_id(0) * rblk, seq_len)
    o_ref[...] = x_ref[...] + w_ref[pl.ds(s0, rblk), :]


import functools


def kernel(x, encoding_weight):
    batch, seq_len, d_model = x.shape
    x2 = x.reshape(batch * seq_len, d_model)
    rblk = 2048
    n_rows = batch * seq_len
    grid = (n_rows // rblk,)
    out = pl.pallas_call(
        functools.partial(_add_kernel, rblk=rblk, seq_len=seq_len),
        grid=grid,
        in_specs=[
            pl.BlockSpec((rblk, d_model), lambda i: (i, 0)),
            pl.BlockSpec((seq_len, d_model), lambda i: (0, 0)),
        ],
        out_specs=pl.BlockSpec((rblk, d_model), lambda i: (i, 0)),
        out_shape=jax.ShapeDtypeStruct((n_rows, d_model), x.dtype),
    )(x2, encoding_weight)
    return out.reshape(batch, seq_len, d_model)


# speedup vs baseline: 3.6737x; 1.0028x over previous
import functools

import jax
import jax.numpy as jnp
from jax import lax
from jax.experimental import pallas as pl


def _add_kernel(x_ref, w_ref, o_ref, *, rblk, seq_len):
    s0 = lax.rem(pl.program_id(0) * rblk, seq_len)
    o_ref[...] = x_ref[...] + w_ref[pl.ds(s0, rblk), :]


def kernel(x, encoding_weight):
    batch, seq_len, d_model = x.shape
    x2 = x.reshape(batch * seq_len, d_model)
    rblk = 2048
    n_rows = batch * seq_len
    grid = (n_rows // rblk,)
    out = pl.pallas_call(
        functools.partial(_add_kernel, rblk=rblk, seq_len=seq_len),
        grid=grid,
        in_specs=[
            pl.BlockSpec((rblk, d_model), lambda i: (i, 0)),
            pl.BlockSpec((seq_len, d_model), lambda i: (0, 0)),
        ],
        out_specs=pl.BlockSpec((rblk, d_model), lambda i: (i, 0)),
        out_shape=jax.ShapeDtypeStruct((n_rows, d_model), x.dtype),
    )(x2, encoding_weight)
    return out.reshape(batch, seq_len, d_model)
